# Initial kernel scaffold; baseline (speedup 1.0000x reference)
import jax
import jax.numpy as jnp
from jax.experimental import pallas as pl


def kernel(x, target):
    raise NotImplementedError("write your pallas kernel here")


# speedup vs baseline: 1.7689x; 1.7689x over previous
import math

import jax
import jax.numpy as jnp
from jax.experimental import pallas as pl
from jax.experimental.pallas import tpu as pltpu

_VOCAB = 100000
_SMOOTHING = 0.1
_CONF = 1.0 - _SMOOTHING
_EPS = _SMOOTHING / (_VOCAB - 2)
_C1 = (_VOCAB - 2) * _EPS * math.log(_EPS) + _CONF * math.log(_CONF)

_N = 1024
_BC = 2048


def _kl_body(x_ref, t_ref, o_ref):
    j = pl.program_id(0)

    @pl.when(j == 0)
    def _init():
        nonpad = jnp.sum((t_ref[:, :] != 0).astype(jnp.float32))
        o_ref[0, 0] = nonpad * _C1

    tgt = t_ref[:, :]
    col0 = j * _BC
    cols = col0 + jax.lax.broadcasted_iota(jnp.int32, (_N, _BC), 1)
    xb = x_ref[:, :]
    w = jnp.where(cols == tgt, _CONF, _EPS)
    valid = (cols > 0) & (cols < _VOCAB) & (tgt != 0)
    contrib = jnp.where(valid, w * xb, 0.0)
    o_ref[0, 0] -= jnp.sum(contrib)


def kernel(x, target):
    n, v = x.shape
    tgt2 = target.astype(jnp.int32).reshape(n, 1)
    nblocks = pl.cdiv(v, _BC)
    out = pl.pallas_call(
        _kl_body,
        grid=(nblocks,),
        in_specs=[
            pl.BlockSpec((n, _BC), lambda j: (0, j)),
            pl.BlockSpec((n, 1), lambda j: (0, 0)),
        ],
        out_specs=pl.BlockSpec(memory_space=pltpu.SMEM),
        out_shape=jax.ShapeDtypeStruct((1, 1), jnp.float32),
    )(x, tgt2)
    return out[0, 0]
